# baseline (speedup 1.0000x reference)
import functools

import jax
import jax.numpy as jnp
from jax import lax
from jax.experimental import pallas as pl
from jax.experimental.pallas import tpu as pltpu
from jax.experimental.pallas import tpu_sc as plsc

N_NODES = 100000
D = 128
OUT = 128
B = 4096
S = 10
L1 = B * (S + 1)
NC, NS = 2, 16
NW = NC * NS
ROWS_PER_W = L1 // NW
C = 32
CHUNKS = ROWS_PER_W // C
NBUF = 4
DEPTH = 3


def _sc_gather_sum(idx_hbm, table_hbm):
    mesh = plsc.VectorSubcoreMesh(core_axis_name="c", subcore_axis_name="s")

    @functools.partial(
        pl.kernel,
        mesh=mesh,
        out_type=(jax.ShapeDtypeStruct((L1, D), jnp.float32),
                  jax.ShapeDtypeStruct((L1, D), jnp.float32)),
        scratch_types=[
            pltpu.VMEM((CHUNKS, S + 1, C), jnp.int32),
        ] + [pltpu.VMEM((C, D), jnp.float32)] * NBUF
          + [pltpu.VMEM((C, D), jnp.float32)] * NBUF
          + [pltpu.SemaphoreType.DMA] * (2 * NBUF),
    )
    def k(idx_h, table_h, self_h, agg_h, idx_all,
          a0, a1, a2, a3, f0, f1, f2, f3,
          sg0, sg1, sg2, sg3, so0, so1, so2, so3):
        acc = [a0, a1, a2, a3]
        slf = [f0, f1, f2, f3]
        sg = [sg0, sg1, sg2, sg3]
        so = [so0, so1, so2, so3]
        wid = lax.axis_index("s") * NC + lax.axis_index("c")
        zeros16 = jnp.zeros((16,), jnp.float32)

        def chunk_start(c, b):
            for i in range(C):
                for v in range(D // 16):
                    acc[b][i, pl.ds(v * 16, 16)] = zeros16
            pltpu.async_copy(table_h.at[idx_all.at[c, 0]], slf[b], sg[b])
            for j in range(1, S + 1):
                pltpu.async_copy(table_h.at[idx_all.at[c, j]], acc[b], sg[b],
                                 add=True)

        def chunk_wait(b):
            for _ in range(S + 1):
                pltpu.make_async_copy(
                    table_h.at[idx_all.at[0, 0]], acc[b], sg[b]).wait()

        def out_start(c, b):
            row = wid * ROWS_PER_W + c * C
            pltpu.async_copy(slf[b], self_h.at[pl.ds(row, C)], so[b])
            pltpu.async_copy(acc[b], agg_h.at[pl.ds(row, C)], so[b])

        def out_wait(b):
            for _ in range(2):
                pltpu.make_async_copy(acc[b], agg_h.at[pl.ds(0, C)],
                                      so[b]).wait()

        pltpu.sync_copy(idx_h.at[wid], idx_all)
        for c in range(DEPTH):
            chunk_start(c, c)

        def body(kk, _):
            for bu in range(NBUF):
                c = kk * NBUF + bu
                chunk_wait(bu)
                out_start(c, bu)
                c2 = c + DEPTH
                b2 = (bu + DEPTH) % NBUF

                @pl.when(c2 < CHUNKS)
                def _():
                    @pl.when(c2 >= NBUF)
                    def _():
                        out_wait(b2)

                    chunk_start(c2, b2)

            return 0

        lax.fori_loop(0, CHUNKS // NBUF, body, 0)
        for b in range(NBUF):
            out_wait(b)

    return k(idx_hbm, table_hbm)


def _tc_dense_body(self_ref, nsum_ref, w1s_ref, w1a_ref, w2s_ref, w2a_ref,
                   out_ref):
    BLK = self_ref.shape[1]
    s1 = self_ref[...].reshape((S + 1) * BLK, D)
    n1 = nsum_ref[...].reshape((S + 1) * BLK, D)
    h1 = jnp.clip(
        jnp.dot(s1, w1s_ref[...], preferred_element_type=jnp.float32)
        + jnp.dot(n1, w1a_ref[...], preferred_element_type=jnp.float32),
        0.0, 6.0)
    h1g = h1.reshape(S + 1, BLK, OUT)
    self2 = h1g[0]
    agg2 = h1g[1]
    for r in range(2, S + 1):
        agg2 = agg2 + h1g[r]
    out_ref[...] = jnp.clip(
        jnp.dot(self2, w2s_ref[...], preferred_element_type=jnp.float32)
        + jnp.dot(agg2, w2a_ref[...], preferred_element_type=jnp.float32),
        0.0, 6.0)


def _tc_dense(self_rows, nsum_rows, w1s, w1a, w2s, w2a):
    BLK = 256
    grid = (B // BLK,)
    return pl.pallas_call(
        _tc_dense_body,
        grid=grid,
        in_specs=[
            pl.BlockSpec((S + 1, BLK, D), lambda i: (0, i, 0)),
            pl.BlockSpec((S + 1, BLK, D), lambda i: (0, i, 0)),
            pl.BlockSpec((D, OUT), lambda i: (0, 0)),
            pl.BlockSpec((D, OUT), lambda i: (0, 0)),
            pl.BlockSpec((OUT, OUT), lambda i: (0, 0)),
            pl.BlockSpec((OUT, OUT), lambda i: (0, 0)),
        ],
        out_specs=pl.BlockSpec((BLK, OUT), lambda i: (i, 0)),
        out_shape=jax.ShapeDtypeStruct((B, OUT), jnp.float32),
    )(self_rows.reshape(S + 1, B, D), nsum_rows.reshape(S + 1, B, D),
      w1s, w1a, w2s, w2a)


def kernel(nodes_batch, neigh_l2, neigh_l1, raw_features, W1, W2):
    nodes_l1 = jnp.concatenate(
        [nodes_batch[:, None], neigh_l2], axis=1).reshape(-1)
    idx11 = jnp.concatenate(
        [nodes_l1[:, None], neigh_l1], axis=1).astype(jnp.int32)
    idx = (idx11.reshape(B, S + 1, S + 1).transpose(1, 0, 2)
           .reshape(NW, CHUNKS, C, S + 1).transpose(0, 1, 3, 2))

    self_rows, nsum_rows = _sc_gather_sum(idx, raw_features)

    inv = jnp.float32(1.0 / S)
    w1s = W1[:, :D].T
    w1a = W1[:, D:].T * inv
    w2s = W2[:, :OUT].T
    w2a = W2[:, OUT:].T * inv
    return _tc_dense(self_rows, nsum_rows, w1s, w1a, w2s, w2a)


# speedup vs baseline: 1.1986x; 1.1986x over previous
import functools

import jax
import jax.numpy as jnp
from jax import lax
from jax.experimental import pallas as pl
from jax.experimental.pallas import tpu as pltpu
from jax.experimental.pallas import tpu_sc as plsc

N_NODES = 100000
D = 128
OUT = 128
B = 4096
S = 10
L1 = B * (S + 1)
NC, NS = 2, 16
NW = NC * NS
ROWS_PER_W = L1 // NW
C = 16
CHUNKS = ROWS_PER_W // C
NBUF = 8
DEPTH = 3
IDEPTH = 5


SUP = 128
CPS = SUP // C
SUPERS = ROWS_PER_W // SUP


def _sc_gather_sum(selfT_hbm, neighT_hbm, table_hbm):
    mesh = plsc.VectorSubcoreMesh(core_axis_name="c", subcore_axis_name="s")

    @functools.partial(
        pl.kernel,
        mesh=mesh,
        out_type=(jax.ShapeDtypeStruct((L1, D), jnp.float32),
                  jax.ShapeDtypeStruct((L1, D), jnp.float32)),
        scratch_types=[pltpu.VMEM((SUP,), jnp.int32)] * 2
          + [pltpu.VMEM((S, SUP), jnp.int32)] * 2
          + [pltpu.VMEM((C, D), jnp.float32)] * NBUF
          + [pltpu.VMEM((C, D), jnp.float32)] * NBUF
          + [pltpu.SemaphoreType.DMA] * 2
          + [pltpu.SemaphoreType.DMA] * (2 * NBUF),
    )
    def k(selfT_h, neighT_h, table_h, self_h, agg_h, *bufs):
        sfx = list(bufs[0:2])
        nblk = list(bufs[2:4])
        acc = list(bufs[4:4 + NBUF])
        slf = list(bufs[4 + NBUF:4 + 2 * NBUF])
        si = list(bufs[4 + 2 * NBUF:6 + 2 * NBUF])
        sg = list(bufs[6 + 2 * NBUF:6 + 3 * NBUF])
        so = list(bufs[6 + 3 * NBUF:6 + 4 * NBUF])
        wid = lax.axis_index("s") * NC + lax.axis_index("c")
        r0w = wid * ROWS_PER_W
        zeros16 = jnp.zeros((16,), jnp.float32)

        def iblk_start(s, ib):
            r0 = r0w + s * SUP
            j2 = r0 // B
            b0 = r0 % B
            pltpu.async_copy(selfT_h.at[j2, pl.ds(b0, SUP)], sfx[ib], si[ib])
            pltpu.async_copy(neighT_h.at[j2, pl.ds(0, S), pl.ds(b0, SUP)],
                             nblk[ib], si[ib])

        def iblk_wait(ib):
            pltpu.make_async_copy(selfT_h.at[0, pl.ds(0, SUP)], sfx[ib],
                                  si[ib]).wait()
            pltpu.make_async_copy(neighT_h.at[0, pl.ds(0, S), pl.ds(0, SUP)],
                                  nblk[ib], si[ib]).wait()

        def chunk_start(c, b, ib, sub):
            for i in range(C):
                for v in range(D // 16):
                    acc[b][i, pl.ds(v * 16, 16)] = zeros16
            pltpu.async_copy(table_h.at[sfx[ib].at[pl.ds(sub * C, C)]],
                             slf[b], sg[b])
            for j in range(S):
                pltpu.async_copy(
                    table_h.at[nblk[ib].at[j, pl.ds(sub * C, C)]],
                    acc[b], sg[b], add=True)

        def chunk_wait(b):
            for _ in range(S + 1):
                pltpu.make_async_copy(
                    table_h.at[sfx[0].at[pl.ds(0, C)]], acc[b], sg[b]).wait()

        def out_start(c, b):
            row = r0w + c * C
            pltpu.async_copy(slf[b], self_h.at[pl.ds(row, C)], so[b])
            pltpu.async_copy(acc[b], agg_h.at[pl.ds(row, C)], so[b])

        def out_wait(b):
            for _ in range(2):
                pltpu.make_async_copy(acc[b], agg_h.at[pl.ds(0, C)],
                                      so[b]).wait()

        def super_body(s, sb):
            @pl.when(s + 1 < SUPERS)
            def _():
                iblk_start(s + 1, 1 - sb)

            for bu in range(CPS):
                c = s * CPS + bu
                chunk_wait(bu)
                out_start(c, bu)
                c2 = c + DEPTH
                b2 = (bu + DEPTH) % NBUF
                sub2 = (bu + DEPTH) % CPS
                ib2 = sb if bu + DEPTH < CPS else 1 - sb
                if bu + DEPTH == CPS:
                    @pl.when(c2 < CHUNKS)
                    def _():
                        iblk_wait(ib2)

                @pl.when(c2 < CHUNKS)
                def _():
                    @pl.when(c2 >= NBUF)
                    def _():
                        out_wait(b2)

                    chunk_start(c2, b2, ib2, sub2)

        iblk_start(0, 0)
        iblk_wait(0)
        for c in range(DEPTH):
            chunk_start(c, c, 0, c)

        super_body(0, 0)

        def body(kk, _):
            super_body(2 * kk + 1, 1)
            super_body(2 * kk + 2, 0)
            return 0

        lax.fori_loop(0, (SUPERS - 1) // 2, body, 0)
        for b in range(NBUF):
            out_wait(b)

    return k(selfT_hbm, neighT_hbm, table_hbm)


def _tc_dense_body(self_ref, nsum_ref, w1s_ref, w1a_ref, w2s_ref, w2a_ref,
                   out_ref):
    BLK = self_ref.shape[1]
    s1 = self_ref[...].reshape((S + 1) * BLK, D)
    n1 = nsum_ref[...].reshape((S + 1) * BLK, D)
    h1 = jnp.clip(
        jnp.dot(s1, w1s_ref[...], preferred_element_type=jnp.float32)
        + jnp.dot(n1, w1a_ref[...], preferred_element_type=jnp.float32),
        0.0, 6.0)
    h1g = h1.reshape(S + 1, BLK, OUT)
    self2 = h1g[0]
    agg2 = h1g[1]
    for r in range(2, S + 1):
        agg2 = agg2 + h1g[r]
    out_ref[...] = jnp.clip(
        jnp.dot(self2, w2s_ref[...], preferred_element_type=jnp.float32)
        + jnp.dot(agg2, w2a_ref[...], preferred_element_type=jnp.float32),
        0.0, 6.0)


def _tc_dense(self_rows, nsum_rows, w1s, w1a, w2s, w2a):
    BLK = 256
    grid = (B // BLK,)
    return pl.pallas_call(
        _tc_dense_body,
        grid=grid,
        in_specs=[
            pl.BlockSpec((S + 1, BLK, D), lambda i: (0, i, 0)),
            pl.BlockSpec((S + 1, BLK, D), lambda i: (0, i, 0)),
            pl.BlockSpec((D, OUT), lambda i: (0, 0)),
            pl.BlockSpec((D, OUT), lambda i: (0, 0)),
            pl.BlockSpec((OUT, OUT), lambda i: (0, 0)),
            pl.BlockSpec((OUT, OUT), lambda i: (0, 0)),
        ],
        out_specs=pl.BlockSpec((BLK, OUT), lambda i: (i, 0)),
        out_shape=jax.ShapeDtypeStruct((B, OUT), jnp.float32),
    )(self_rows.reshape(S + 1, B, D), nsum_rows.reshape(S + 1, B, D),
      w1s, w1a, w2s, w2a)


def kernel(nodes_batch, neigh_l2, neigh_l1, raw_features, W1, W2):
    selfT = jnp.concatenate(
        [nodes_batch[:, None], neigh_l2], axis=1).astype(jnp.int32).T
    neighT = (neigh_l1.astype(jnp.int32).reshape(B, (S + 1) * S).T
              .reshape(S + 1, S, B))

    self_rows, nsum_rows = _sc_gather_sum(selfT, neighT, raw_features)

    inv = jnp.float32(1.0 / S)
    w1s = W1[:, :D].T
    w1a = W1[:, D:].T * inv
    w2s = W2[:, :OUT].T
    w2a = W2[:, OUT:].T * inv
    return _tc_dense(self_rows, nsum_rows, w1s, w1a, w2s, w2a)


# speedup vs baseline: 1.2310x; 1.0271x over previous
import functools

import jax
import jax.numpy as jnp
from jax import lax
from jax.experimental import pallas as pl
from jax.experimental.pallas import tpu as pltpu
from jax.experimental.pallas import tpu_sc as plsc

N_NODES = 100000
D = 128
OUT = 128
B = 4096
S = 10
L1 = B * (S + 1)
NC, NS = 2, 16
NW = NC * NS
ROWS_PER_W = L1 // NW
C = 16
CHUNKS = ROWS_PER_W // C
NBUF = 8
DEPTH = 3
IDEPTH = 5


SUP = 128
CPS = SUP // C
SUPERS = ROWS_PER_W // SUP


def _sc_gather_sum(selfT_hbm, neighT_hbm, table_hbm):
    mesh = plsc.VectorSubcoreMesh(core_axis_name="c", subcore_axis_name="s")

    @functools.partial(
        pl.kernel,
        mesh=mesh,
        out_type=(jax.ShapeDtypeStruct((L1, D), jnp.float32),
                  jax.ShapeDtypeStruct((L1, D), jnp.float32)),
        scratch_types=[pltpu.VMEM((SUP,), jnp.int32)] * 2
          + [pltpu.VMEM((S, SUP), jnp.int32)] * 2
          + [pltpu.VMEM((C, D), jnp.float32)] * NBUF
          + [pltpu.VMEM((C, D), jnp.float32)] * NBUF
          + [pltpu.SemaphoreType.DMA] * 2
          + [pltpu.SemaphoreType.DMA] * (2 * NBUF),
    )
    def k(selfT_h, neighT_h, table_h, self_h, agg_h, *bufs):
        sfx = list(bufs[0:2])
        nblk = list(bufs[2:4])
        acc = list(bufs[4:4 + NBUF])
        slf = list(bufs[4 + NBUF:4 + 2 * NBUF])
        si = list(bufs[4 + 2 * NBUF:6 + 2 * NBUF])
        sg = list(bufs[6 + 2 * NBUF:6 + 3 * NBUF])
        so = list(bufs[6 + 3 * NBUF:6 + 4 * NBUF])
        wid = lax.axis_index("s") * NC + lax.axis_index("c")
        r0w = wid * ROWS_PER_W
        zeros16 = jnp.zeros((16,), jnp.float32)

        def iblk_start(s, ib):
            r0 = r0w + s * SUP
            j2 = r0 // B
            b0 = r0 % B
            pltpu.async_copy(selfT_h.at[j2, pl.ds(b0, SUP)], sfx[ib], si[ib])
            pltpu.async_copy(neighT_h.at[j2, pl.ds(0, S), pl.ds(b0, SUP)],
                             nblk[ib], si[ib])

        def iblk_wait(ib):
            pltpu.make_async_copy(selfT_h.at[0, pl.ds(0, SUP)], sfx[ib],
                                  si[ib]).wait()
            pltpu.make_async_copy(neighT_h.at[0, pl.ds(0, S), pl.ds(0, SUP)],
                                  nblk[ib], si[ib]).wait()

        def chunk_start(c, b, ib, sub):
            for i in range(C):
                for v in range(D // 16):
                    acc[b][i, pl.ds(v * 16, 16)] = zeros16
            pltpu.async_copy(table_h.at[sfx[ib].at[pl.ds(sub * C, C)]],
                             slf[b], sg[b])
            for j in range(S):
                pltpu.async_copy(
                    table_h.at[nblk[ib].at[j, pl.ds(sub * C, C)]],
                    acc[b], sg[b], add=True)

        def chunk_wait(b):
            for _ in range(S + 1):
                pltpu.make_async_copy(
                    table_h.at[sfx[0].at[pl.ds(0, C)]], acc[b], sg[b]).wait()

        def out_start(c, b):
            row = r0w + c * C
            pltpu.async_copy(slf[b], self_h.at[pl.ds(row, C)], so[b])
            pltpu.async_copy(acc[b], agg_h.at[pl.ds(row, C)], so[b])

        def out_wait(b):
            for _ in range(2):
                pltpu.make_async_copy(acc[b], agg_h.at[pl.ds(0, C)],
                                      so[b]).wait()

        def super_body(s, sb):
            @pl.when(s + 1 < SUPERS)
            def _():
                iblk_start(s + 1, 1 - sb)

            for bu in range(CPS):
                c = s * CPS + bu
                chunk_wait(bu)
                out_start(c, bu)
                c2 = c + DEPTH
                b2 = (bu + DEPTH) % NBUF
                sub2 = (bu + DEPTH) % CPS
                ib2 = sb if bu + DEPTH < CPS else 1 - sb
                if bu + DEPTH == CPS:
                    @pl.when(c2 < CHUNKS)
                    def _():
                        iblk_wait(ib2)

                @pl.when(c2 < CHUNKS)
                def _():
                    @pl.when(c2 >= NBUF)
                    def _():
                        out_wait(b2)

                    chunk_start(c2, b2, ib2, sub2)

        iblk_start(0, 0)
        iblk_wait(0)
        for c in range(DEPTH):
            chunk_start(c, c, 0, c)

        super_body(0, 0)

        def body(kk, _):
            super_body(2 * kk + 1, 1)
            super_body(2 * kk + 2, 0)
            return 0

        lax.fori_loop(0, (SUPERS - 1) // 2, body, 0)
        for b in range(NBUF):
            out_wait(b)

    return k(selfT_hbm, neighT_hbm, table_hbm)


def _tc_dense_body(self_ref, nsum_ref, w1s_ref, w1a_ref, w2s_ref, w2a_ref,
                   out_ref):
    BLK = self_ref.shape[1]
    s1 = self_ref[...].reshape((S + 1) * BLK, D)
    n1 = nsum_ref[...].reshape((S + 1) * BLK, D)
    h1 = jnp.clip(
        jnp.dot(s1, w1s_ref[...], preferred_element_type=jnp.float32)
        + jnp.dot(n1, w1a_ref[...], preferred_element_type=jnp.float32),
        0.0, 6.0)
    h1g = h1.reshape(S + 1, BLK, OUT)
    self2 = h1g[0]
    agg2 = h1g[1]
    for r in range(2, S + 1):
        agg2 = agg2 + h1g[r]
    out_ref[...] = jnp.clip(
        jnp.dot(self2, w2s_ref[...], preferred_element_type=jnp.float32)
        + jnp.dot(agg2, w2a_ref[...], preferred_element_type=jnp.float32),
        0.0, 6.0)


def _tc_dense(self_rows, nsum_rows, w1s, w1a, w2s, w2a):
    BLK = 512
    grid = (B // BLK,)
    return pl.pallas_call(
        _tc_dense_body,
        grid=grid,
        in_specs=[
            pl.BlockSpec((S + 1, BLK, D), lambda i: (0, i, 0)),
            pl.BlockSpec((S + 1, BLK, D), lambda i: (0, i, 0)),
            pl.BlockSpec((D, OUT), lambda i: (0, 0)),
            pl.BlockSpec((D, OUT), lambda i: (0, 0)),
            pl.BlockSpec((OUT, OUT), lambda i: (0, 0)),
            pl.BlockSpec((OUT, OUT), lambda i: (0, 0)),
        ],
        out_specs=pl.BlockSpec((BLK, OUT), lambda i: (i, 0)),
        out_shape=jax.ShapeDtypeStruct((B, OUT), jnp.float32),
    )(self_rows.reshape(S + 1, B, D), nsum_rows.reshape(S + 1, B, D),
      w1s, w1a, w2s, w2a)


def kernel(nodes_batch, neigh_l2, neigh_l1, raw_features, W1, W2):
    selfT = jnp.concatenate(
        [nodes_batch[:, None], neigh_l2], axis=1).astype(jnp.int32).T
    neighT = (neigh_l1.astype(jnp.int32).reshape(B, (S + 1) * S).T
              .reshape(S + 1, S, B))

    self_rows, nsum_rows = _sc_gather_sum(selfT, neighT, raw_features)

    inv = jnp.float32(1.0 / S)
    w1s = W1[:, :D].T
    w1a = W1[:, D:].T * inv
    w2s = W2[:, :OUT].T
    w2a = W2[:, OUT:].T * inv
    return _tc_dense(self_rows, nsum_rows, w1s, w1a, w2s, w2a)


# speedup vs baseline: 1.2352x; 1.0034x over previous
import functools

import jax
import jax.numpy as jnp
from jax import lax
from jax.experimental import pallas as pl
from jax.experimental.pallas import tpu as pltpu
from jax.experimental.pallas import tpu_sc as plsc

N_NODES = 100000
D = 128
OUT = 128
B = 4096
S = 10
L1 = B * (S + 1)
NC, NS = 2, 16
NW = NC * NS
ROWS_PER_W = L1 // NW
C = 16
CHUNKS = ROWS_PER_W // C
NBUF = 8
DEPTH = 3
IDEPTH = 5


SUP = 128
CPS = SUP // C
SUPERS = ROWS_PER_W // SUP


def _sc_gather_sum(selfT_hbm, neighT_hbm, table_hbm):
    mesh = plsc.VectorSubcoreMesh(core_axis_name="c", subcore_axis_name="s")

    @functools.partial(
        pl.kernel,
        mesh=mesh,
        out_type=(jax.ShapeDtypeStruct((L1, D), jnp.float32),
                  jax.ShapeDtypeStruct((L1, D), jnp.float32)),
        scratch_types=[pltpu.VMEM((SUP,), jnp.int32)] * 2
          + [pltpu.VMEM((S, SUP), jnp.int32)] * 2
          + [pltpu.VMEM((C, D), jnp.float32)] * NBUF
          + [pltpu.VMEM((C, D), jnp.float32)] * NBUF
          + [pltpu.SemaphoreType.DMA] * 2
          + [pltpu.SemaphoreType.DMA] * (2 * NBUF),
    )
    def k(selfT_h, neighT_h, table_h, self_h, agg_h, *bufs):
        sfx = list(bufs[0:2])
        nblk = list(bufs[2:4])
        acc = list(bufs[4:4 + NBUF])
        slf = list(bufs[4 + NBUF:4 + 2 * NBUF])
        si = list(bufs[4 + 2 * NBUF:6 + 2 * NBUF])
        sg = list(bufs[6 + 2 * NBUF:6 + 3 * NBUF])
        so = list(bufs[6 + 3 * NBUF:6 + 4 * NBUF])
        wid = lax.axis_index("s") * NC + lax.axis_index("c")
        r0w = wid * ROWS_PER_W
        zeros16 = jnp.zeros((16,), jnp.float32)

        def iblk_start(s, ib):
            r0 = r0w + s * SUP
            j2 = r0 // B
            b0 = r0 % B
            pltpu.async_copy(selfT_h.at[j2, pl.ds(b0, SUP)], sfx[ib], si[ib])
            pltpu.async_copy(neighT_h.at[j2, pl.ds(0, S), pl.ds(b0, SUP)],
                             nblk[ib], si[ib])

        def iblk_wait(ib):
            pltpu.make_async_copy(selfT_h.at[0, pl.ds(0, SUP)], sfx[ib],
                                  si[ib]).wait()
            pltpu.make_async_copy(neighT_h.at[0, pl.ds(0, S), pl.ds(0, SUP)],
                                  nblk[ib], si[ib]).wait()

        def chunk_start(c, b, ib, sub):
            for i in range(C):
                for v in range(D // 16):
                    acc[b][i, pl.ds(v * 16, 16)] = zeros16
            pltpu.async_copy(table_h.at[sfx[ib].at[pl.ds(sub * C, C)]],
                             slf[b], sg[b])
            for j in range(S):
                pltpu.async_copy(
                    table_h.at[nblk[ib].at[j, pl.ds(sub * C, C)]],
                    acc[b], sg[b], add=True)

        def chunk_wait(b):
            for _ in range(S + 1):
                pltpu.make_async_copy(
                    table_h.at[sfx[0].at[pl.ds(0, C)]], acc[b], sg[b]).wait()

        def out_start(c, b):
            row = r0w + c * C
            pltpu.async_copy(slf[b], self_h.at[pl.ds(row, C)], so[b])
            pltpu.async_copy(acc[b], agg_h.at[pl.ds(row, C)], so[b])

        def out_wait(b):
            for _ in range(2):
                pltpu.make_async_copy(acc[b], agg_h.at[pl.ds(0, C)],
                                      so[b]).wait()

        def super_body(s, sb):
            @pl.when(s + 1 < SUPERS)
            def _():
                iblk_start(s + 1, 1 - sb)

            for bu in range(CPS):
                c = s * CPS + bu
                chunk_wait(bu)
                out_start(c, bu)
                c2 = c + DEPTH
                b2 = (bu + DEPTH) % NBUF
                sub2 = (bu + DEPTH) % CPS
                ib2 = sb if bu + DEPTH < CPS else 1 - sb
                if bu + DEPTH == CPS:
                    @pl.when(c2 < CHUNKS)
                    def _():
                        iblk_wait(ib2)

                @pl.when(c2 < CHUNKS)
                def _():
                    @pl.when(c2 >= NBUF)
                    def _():
                        out_wait(b2)

                    chunk_start(c2, b2, ib2, sub2)

        iblk_start(0, 0)
        iblk_wait(0)
        for c in range(DEPTH):
            chunk_start(c, c, 0, c)

        super_body(0, 0)

        def body(kk, _):
            super_body(2 * kk + 1, 1)
            super_body(2 * kk + 2, 0)
            return 0

        lax.fori_loop(0, (SUPERS - 1) // 2, body, 0)
        for b in range(NBUF):
            out_wait(b)

    return k(selfT_hbm, neighT_hbm, table_hbm)


def _tc_dense_body(self_ref, nsum_ref, w1s_ref, w1a_ref, w2s_ref, w2a_ref,
                   out_ref):
    BLK = self_ref.shape[1]
    s1 = self_ref[...].reshape((S + 1) * BLK, D)
    n1 = nsum_ref[...].reshape((S + 1) * BLK, D)
    h1 = jnp.clip(
        jnp.dot(s1, w1s_ref[...], preferred_element_type=jnp.float32)
        + jnp.dot(n1, w1a_ref[...], preferred_element_type=jnp.float32),
        0.0, 6.0)
    h1g = h1.reshape(S + 1, BLK, OUT)
    self2 = h1g[0]
    agg2 = h1g[1]
    for r in range(2, S + 1):
        agg2 = agg2 + h1g[r]
    out_ref[...] = jnp.clip(
        jnp.dot(self2, w2s_ref[...], preferred_element_type=jnp.float32)
        + jnp.dot(agg2, w2a_ref[...], preferred_element_type=jnp.float32),
        0.0, 6.0)


def _tc_dense(self_rows, nsum_rows, w1s, w1a, w2s, w2a):
    BLK = 1024
    grid = (B // BLK,)
    return pl.pallas_call(
        _tc_dense_body,
        grid=grid,
        in_specs=[
            pl.BlockSpec((S + 1, BLK, D), lambda i: (0, i, 0)),
            pl.BlockSpec((S + 1, BLK, D), lambda i: (0, i, 0)),
            pl.BlockSpec((D, OUT), lambda i: (0, 0)),
            pl.BlockSpec((D, OUT), lambda i: (0, 0)),
            pl.BlockSpec((OUT, OUT), lambda i: (0, 0)),
            pl.BlockSpec((OUT, OUT), lambda i: (0, 0)),
        ],
        out_specs=pl.BlockSpec((BLK, OUT), lambda i: (i, 0)),
        out_shape=jax.ShapeDtypeStruct((B, OUT), jnp.float32),
    )(self_rows.reshape(S + 1, B, D), nsum_rows.reshape(S + 1, B, D),
      w1s, w1a, w2s, w2a)


def kernel(nodes_batch, neigh_l2, neigh_l1, raw_features, W1, W2):
    selfT = jnp.concatenate(
        [nodes_batch[:, None], neigh_l2], axis=1).astype(jnp.int32).T
    neighT = (neigh_l1.astype(jnp.int32).reshape(B, (S + 1) * S).T
              .reshape(S + 1, S, B))

    self_rows, nsum_rows = _sc_gather_sum(selfT, neighT, raw_features)

    inv = jnp.float32(1.0 / S)
    w1s = W1[:, :D].T
    w1a = W1[:, D:].T * inv
    w2s = W2[:, :OUT].T
    w2a = W2[:, OUT:].T * inv
    return _tc_dense(self_rows, nsum_rows, w1s, w1a, w2s, w2a)
